# Initial kernel scaffold; baseline (speedup 1.0000x reference)
import jax
import jax.numpy as jnp
from jax.experimental import pallas as pl


def kernel(location_x, loc_table, user_table, time_table):
    raise NotImplementedError("write your pallas kernel here")


# speedup vs baseline: 1.0108x; 1.0108x over previous
import functools

import numpy as np
import jax
import jax.numpy as jnp
from jax import lax
from jax.experimental import pallas as pl
from jax.experimental.pallas import tpu as pltpu
from jax.experimental.pallas import tpu_sc as plsc

NUM_LOCATIONS = 1000000
BASE_DIM = 64
BANDWIDTH = 0.5
BATCH = 4096
SEQ_LEN = 50

N_IDX = BATCH * SEQ_LEN
NUM_CORES = 2
NUM_SUBCORES = 16
NW = NUM_CORES * NUM_SUBCORES
PER_W = N_IDX // NW
CHUNK = 128
NCHUNK = PER_W // CHUNK

_t = np.arange(24, dtype=np.float32)
_absdiff = np.abs(_t[None, :] - _t[:, None])
_dist = np.minimum(_absdiff, 24.0 - _absdiff)
_W_SMOOTH = np.exp(-0.5 * (_dist / BANDWIDTH) ** 2).astype(np.float32)

_sc_mesh = plsc.VectorSubcoreMesh(core_axis_name="c", subcore_axis_name="s")


@functools.partial(
    pl.kernel,
    out_type=jax.ShapeDtypeStruct((N_IDX, BASE_DIM), jnp.float32),
    mesh=_sc_mesh,
    scratch_types=[
        pltpu.VMEM((NCHUNK, CHUNK), jnp.int32),
        pltpu.VMEM((2, CHUNK, BASE_DIM), jnp.float32),
        pltpu.SemaphoreType.DMA,
    ],
    compiler_params=pltpu.CompilerParams(use_tc_tiling_on_sc=False),
)
def _sc_gather(idx_hbm, table_hbm, out_hbm, idx_v, rows_v, gsem):
    wid = lax.axis_index("s") * NUM_CORES + lax.axis_index("c")
    base = wid * PER_W
    pltpu.sync_copy(idx_hbm.at[wid], idx_v)

    def body(j, _):
        pltpu.async_copy(table_hbm.at[idx_v.at[j]], rows_v.at[0], gsem).wait()
        pltpu.sync_copy(rows_v.at[0], out_hbm.at[pl.ds(base + j * CHUNK, CHUNK)])
        return ()

    lax.fori_loop(0, NCHUNK, body, ())


def _smooth_body(w_ref, t_ref, o_ref):
    o_ref[...] = jnp.dot(w_ref[...], t_ref[...],
                         preferred_element_type=jnp.float32)


def kernel(location_x, loc_table, user_table, time_table):
    idx = location_x.reshape(NW, NCHUNK, CHUNK).astype(jnp.int32)
    loc_flat = _sc_gather(idx, loc_table)
    loc_embedded = loc_flat.reshape(BATCH, SEQ_LEN, BASE_DIM)
    smoothed = pl.pallas_call(
        _smooth_body,
        out_shape=jax.ShapeDtypeStruct((24, BASE_DIM), jnp.float32),
    )(jnp.asarray(_W_SMOOTH), time_table)
    return (loc_embedded, time_table, smoothed, user_table)
